# baseline (speedup 1.0000x reference)
import jax
import jax.numpy as jnp
from jax import lax
from jax.experimental import pallas as pl
from jax.experimental.pallas import tpu as pltpu
from jax.experimental.pallas import tpu_sc as plsc

_NE = 64
_TOK = 32768
_TOPK = 2
_NIDX = _TOK * _TOPK
_NW = 32
_PER_W = _NIDX // _NW
_CHUNKS = _PER_W // 16
_LW = 0.01


def _sc_hist_body(idx_hbm, out_hbm, idx_v, hist_v, counts_v):
    c = lax.axis_index("c")
    s = lax.axis_index("s")
    wid = s * 2 + c
    base = wid * _PER_W
    pltpu.sync_copy(idx_hbm.at[pl.ds(base, _PER_W)], idx_v)

    zeros16 = jnp.zeros((16,), jnp.float32)
    for j in range(16 * _NE // 16):
        hist_v[pl.ds(j * 16, 16)] = zeros16

    lane = lax.iota(jnp.int32, 16)
    ones16 = jnp.ones((16,), jnp.float32)

    def body(i, carry):
        v = idx_v[pl.ds(i * 16, 16)]
        addr = lane * _NE + v
        plsc.addupdate_scatter(hist_v, (addr,), ones16)
        return carry

    lax.fori_loop(0, _CHUNKS, body, 0)

    for k in range(_NE // 16):
        acc = zeros16
        for l in range(16):
            acc = acc + hist_v[pl.ds(l * _NE + k * 16, 16)]
        counts_v[pl.ds(k * 16, 16)] = acc

    pltpu.sync_copy(counts_v, out_hbm.at[pl.ds(wid * _NE, _NE)])


_sc_hist_cached = None


def _sc_hist(idx):
    global _sc_hist_cached
    if _sc_hist_cached is None:
        _sc_hist_cached = pl.kernel(
            _sc_hist_body,
            out_type=jax.ShapeDtypeStruct((_NW * _NE,), jnp.float32),
            mesh=plsc.VectorSubcoreMesh(core_axis_name="c", subcore_axis_name="s"),
            scratch_types=[
                pltpu.VMEM((_PER_W,), jnp.int32),
                pltpu.VMEM((16 * _NE,), jnp.float32),
                pltpu.VMEM((_NE,), jnp.float32),
            ],
            compiler_params=pltpu.CompilerParams(needs_layout_passes=False),
        )
    return _sc_hist_cached(idx)


_BT = 1024
_GRID = _TOK // _BT


def _tc_loss_body(x_ref, h_ref, out_ref, acc_ref):
    pid = pl.program_id(0)

    @pl.when(pid == 0)
    def _():
        acc_ref[...] = jnp.zeros_like(acc_ref)

    x = x_ref[...]
    m = jnp.max(x, axis=1, keepdims=True)
    e = jnp.exp(x - m)
    s = jnp.sum(e, axis=1, keepdims=True)
    p = e / s
    acc_ref[0:1, :] += jnp.sum(p, axis=0, keepdims=True)

    @pl.when(pid == _GRID - 1)
    def _():
        counts = jnp.sum(h_ref[...], axis=0, keepdims=True)
        d = jnp.sum(acc_ref[0:1, :] * counts)
        x64 = d * (float(_NE) / (float(_TOK) * float(_NIDX))) - 1.0
        out_ref[0, 0] = jnp.maximum(x64, 0.0) * _LW


def _tc_loss(router_logits, hist):
    return pl.pallas_call(
        _tc_loss_body,
        grid=(_GRID,),
        in_specs=[
            pl.BlockSpec((_BT, _NE), lambda i: (i, 0)),
            pl.BlockSpec((_NW, _NE), lambda i: (0, 0)),
        ],
        out_specs=pl.BlockSpec((1, 1), lambda i: (0, 0),
                               memory_space=pltpu.SMEM),
        out_shape=jax.ShapeDtypeStruct((1, 1), jnp.float32),
        scratch_shapes=[pltpu.VMEM((8, _NE), jnp.float32)],
        compiler_params=pltpu.CompilerParams(
            dimension_semantics=("arbitrary",),
        ),
    )(router_logits, hist)


def kernel(router_logits, expert_indices):
    idx = expert_indices.reshape(-1).astype(jnp.int32)
    hist = _sc_hist(idx).reshape(_NW, _NE)
    out = _tc_loss(router_logits, hist)
    return out.reshape(())
